# Initial kernel scaffold; baseline (speedup 1.0000x reference)
import jax
import jax.numpy as jnp
from jax.experimental import pallas as pl


def kernel(features, image_size, conv_w, conv_b, cls_w, cls_b, bbox_w, bbox_b):
    raise NotImplementedError("write your pallas kernel here")


# speedup vs baseline: 1.0001x; 1.0001x over previous
import jax
import jax.numpy as jnp
import numpy as np
from jax import lax
from jax.experimental import pallas as pl

PRE_NMS_TOPK = 600
POST_NMS_TOPK = 100
NMS_THRESH = 0.7


def _gen_anchors(base_size=16, ratios=(0.5, 1, 2), scales=(8, 16, 32)):
    anchors = []
    for scale in scales:
        for ratio in ratios:
            w = base_size * scale * ratio ** 0.5
            h = base_size * scale / ratio ** 0.5
            anchors.append([-w / 2, -h / 2, w / 2, h / 2])
    return jnp.asarray(np.array(anchors, dtype=np.float32))


def _conv2d(x, w, b, pad):
    out = lax.conv_general_dilated(x, w, window_strides=(1, 1), padding=pad,
                                   dimension_numbers=('NCHW', 'OIHW', 'NCHW'))
    return out + b[None, :, None, None]


def kernel(features, image_size, conv_w, conv_b, cls_w, cls_b, bbox_w, bbox_b):
    B, _, H, W = features.shape
    t = jax.nn.relu(_conv2d(features, conv_w, conv_b, 'SAME'))
    logits = _conv2d(t, cls_w, cls_b, 'VALID')
    bbox_deltas = _conv2d(t, bbox_w, bbox_b, 'VALID')
    shift_x = jnp.arange(W) * 16
    shift_y = jnp.arange(H) * 16
    sy, sx = jnp.meshgrid(shift_y, shift_x, indexing='ij')
    shifts = jnp.stack((sx, sy, sx, sy), axis=2).reshape(-1, 4).astype(jnp.float32)
    anchors = _gen_anchors()
    anchors_all = (anchors[None, :, :] + shifts[:, None, :]).reshape(-1, 4)
    outs = []
    for b in range(B):
        scores = jnp.transpose(logits[b], (1, 2, 0)).reshape(-1)
        deltas = jnp.transpose(bbox_deltas[b], (1, 2, 0)).reshape(-1, 4)
        widths = anchors_all[:, 2] - anchors_all[:, 0]
        heights = anchors_all[:, 3] - anchors_all[:, 1]
        ctr_x = anchors_all[:, 0] + 0.5 * widths
        ctr_y = anchors_all[:, 1] + 0.5 * heights
        dx, dy, dw, dh = deltas[:, 0], deltas[:, 1], deltas[:, 2], deltas[:, 3]
        pred_ctr_x = ctr_x + dx * widths
        pred_ctr_y = ctr_y + dy * heights
        pred_w = widths * jnp.exp(dw)
        pred_h = heights * jnp.exp(dh)
        proposals = jnp.stack([pred_ctr_x - 0.5 * pred_w,
                               pred_ctr_y - 0.5 * pred_h,
                               pred_ctr_x + 0.5 * pred_w,
                               pred_ctr_y + 0.5 * pred_h], axis=1)
        px = jnp.clip(proposals[:, 0::2], 0, image_size[1])
        py = jnp.clip(proposals[:, 1::2], 0, image_size[0])
        proposals = jnp.stack([px[:, 0], py[:, 0], px[:, 1], py[:, 1]], axis=1)
        top_scores, top_idx = lax.top_k(scores, PRE_NMS_TOPK)
        proposals = proposals[top_idx]
        boxes = lax.stop_gradient(proposals)
        scores_s = lax.stop_gradient(top_scores)
        order = jnp.argsort(-scores_s)
        bx = boxes[order]
        x1, y1, x2, y2 = bx[:, 0], bx[:, 1], bx[:, 2], bx[:, 3]
        areas = (x2 - x1) * (y2 - y1)
        xx1 = jnp.maximum(x1[:, None], x1[None, :])
        yy1 = jnp.maximum(y1[:, None], y1[None, :])
        xx2 = jnp.minimum(x2[:, None], x2[None, :])
        yy2 = jnp.minimum(y2[:, None], y2[None, :])
        inter = jnp.clip(xx2 - xx1, 0) * jnp.clip(yy2 - yy1, 0)
        union = areas[:, None] + areas[None, :] - inter
        iou = jnp.where(union > 0, inter / jnp.where(union > 0, union, 1.0), 0.0)
        n = PRE_NMS_TOPK
        idxs = jnp.arange(n)

        def body(i, keep):
            suppress = keep[i] & (iou[i] > NMS_THRESH) & (idxs > i)
            return keep & (~suppress)
        keep = lax.fori_loop(0, n, body, jnp.ones(n, dtype=bool))
        pos = jnp.where(keep, jnp.arange(n), n)
        pos_sorted = jnp.sort(pos)[:POST_NMS_TOPK]
        valid = pos_sorted < n
        sel = order[jnp.minimum(pos_sorted, n - 1)]
        kept = proposals[sel] * valid[:, None].astype(proposals.dtype)
        outs.append(kept)
    return jnp.stack(outs, axis=0)


# speedup vs baseline: 8.0554x; 8.0544x over previous
import functools
import numpy as np
import jax
import jax.numpy as jnp
from jax import lax
from jax.experimental import pallas as pl
from jax.experimental.pallas import tpu as pltpu

PRE = 600
POST = 100
THR = 0.7
NSEL = 640
I32MIN = np.int32(-2147483648)
I32TOP = np.int32(0x7FFFFFFF)


def _trunk_body(x_ref, wt_ref, w2_ref, b256_ref, b80_ref, img_ref,
                aw_ref, ah_ref, acx_ref, acy_ref,
                sc_ref, x1_ref, y1_ref, x2_ref, y2_ref):
    X = x_ref[0]
    p = lax.broadcasted_iota(jnp.int32, (1024, 1), 0)
    xcol = p & 31
    acc = jnp.zeros((1024, 256), jnp.float32)
    for dy in range(3):
        for dx in range(3):
            s = (dy - 1) * 32 + (dx - 1)
            if s > 0:
                Xs = jnp.concatenate(
                    [X[s:], jnp.zeros((s, 384), jnp.float32)], axis=0)
            elif s < 0:
                Xs = jnp.concatenate(
                    [jnp.zeros((-s, 384), jnp.float32), X[:s]], axis=0)
            else:
                Xs = X
            if dx == 0:
                Xs = Xs * (xcol >= 1).astype(jnp.float32)
            elif dx == 2:
                Xs = Xs * (xcol <= 30).astype(jnp.float32)
            acc = acc + jnp.dot(Xs, wt_ref[dy * 3 + dx],
                                preferred_element_type=jnp.float32)
    t = jnp.maximum(acc + b256_ref[...], 0.0)
    out80 = jnp.dot(t, w2_ref[...],
                    preferred_element_type=jnp.float32) + b80_ref[...]
    sc = out80[:, 0:9]
    dxv = out80[:, 16:25]
    dyv = out80[:, 32:41]
    dwv = out80[:, 48:57]
    dhv = out80[:, 64:73]
    aw = aw_ref[...]
    ah = ah_ref[...]
    pcx = acx_ref[...] + dxv * aw
    pcy = acy_ref[...] + dyv * ah
    pw = aw * jnp.exp(dwv)
    ph = ah * jnp.exp(dhv)
    imh = img_ref[0].astype(jnp.float32)
    imw = img_ref[1].astype(jnp.float32)
    sc_ref[0] = sc
    x1_ref[0] = jnp.minimum(jnp.maximum(pcx - 0.5 * pw, 0.0), imw)
    y1_ref[0] = jnp.minimum(jnp.maximum(pcy - 0.5 * ph, 0.0), imh)
    x2_ref[0] = jnp.minimum(jnp.maximum(pcx + 0.5 * pw, 0.0), imw)
    y2_ref[0] = jnp.minimum(jnp.maximum(pcy + 0.5 * ph, 0.0), imh)


def _run_trunk(X, Wt, W2, b256, b80, img, aw, ah, acx, acy):
    B = X.shape[0]
    o = jax.ShapeDtypeStruct((B, 1024, 9), jnp.float32)
    full = lambda shp: pl.BlockSpec(shp, lambda b: tuple(0 for _ in shp))
    return pl.pallas_call(
        _trunk_body,
        grid=(B,),
        in_specs=[
            pl.BlockSpec((1, 1024, 384), lambda b: (b, 0, 0)),
            full((9, 384, 256)),
            full((256, 80)),
            full((1, 256)),
            full((1, 80)),
            pl.BlockSpec(memory_space=pltpu.SMEM),
            full((1, 9)),
            full((1, 9)),
            full((1024, 9)),
            full((1024, 9)),
        ],
        out_specs=[pl.BlockSpec((1, 1024, 9), lambda b: (b, 0, 0))] * 5,
        out_shape=[o] * 5,
    )(X, Wt, W2, b256, b80, img, aw, ah, acx, acy)


def _tau_body(sc_ref, tau_ref):
    s = sc_ref[...]
    i = lax.bitcast_convert_type(s, jnp.int32)
    K = i ^ (lax.shift_right_arithmetic(i, 31) & I32TOP)
    U = K ^ I32MIN

    def body(it, v):
        j = 31 - it
        cand = v | lax.shift_left(jnp.int32(1), j)
        m = ~(lax.shift_left(jnp.int32(1), j) - 1)
        a = (U & m) ^ I32MIN
        b = cand ^ I32MIN
        cnt = jnp.sum((a >= b).astype(jnp.int32), axis=1, keepdims=True)
        return jnp.where(cnt >= PRE, cand, v)

    v = lax.fori_loop(0, 32, body,
                      jnp.zeros((s.shape[0], 1), jnp.int32))
    tau_ref[...] = v ^ I32MIN


def _run_tau(scores):
    B = scores.shape[0]
    return pl.pallas_call(
        _tau_body,
        out_shape=jax.ShapeDtypeStruct((B, 1), jnp.int32),
    )(scores)


def _nms_body(sr_ref, scv_ref, ir_ref, icv_ref, bc_ref, br_ref,
              out_ref, s_scr):
    sr = sr_ref[0]
    scv = scv_ref[0]
    ir = ir_ref[0]
    icv = icv_ref[0]
    bc = bc_ref[0]
    br = br_ref[0]

    beats_ji = (sr > scv) | ((sr == scv) & (ir < icv))
    rank_col = jnp.sum(beats_ji.astype(jnp.float32), axis=1, keepdims=True)
    beats_ij = (scv > sr) | ((scv == sr) & (icv < ir))
    rank_row = jnp.sum(beats_ij.astype(jnp.float32), axis=0, keepdims=True)

    r_c = lax.broadcasted_iota(jnp.int32, (NSEL, 1), 0).astype(jnp.float32)
    r_r = lax.broadcasted_iota(jnp.int32, (1, NSEL), 1).astype(jnp.float32)
    P = (r_c == rank_row).astype(jnp.float32)
    PT = (rank_col == r_r).astype(jnp.float32)
    scol = jnp.dot(P, bc, preferred_element_type=jnp.float32)
    srow = jnp.dot(br, PT, preferred_element_type=jnp.float32)

    x1c = scol[:, 0:1]
    y1c = scol[:, 1:2]
    x2c = scol[:, 2:3]
    y2c = scol[:, 3:4]
    x1r = srow[0:1, :]
    y1r = srow[1:2, :]
    x2r = srow[2:3, :]
    y2r = srow[3:4, :]
    areas_c = (x2c - x1c) * (y2c - y1c)
    areas_r = (x2r - x1r) * (y2r - y1r)
    xx1 = jnp.maximum(x1c, x1r)
    yy1 = jnp.maximum(y1c, y1r)
    xx2 = jnp.minimum(x2c, x2r)
    yy2 = jnp.minimum(y2c, y2r)
    inter = jnp.maximum(xx2 - xx1, 0.0) * jnp.maximum(yy2 - yy1, 0.0)
    union = areas_c + areas_r - inter
    iou = jnp.where(union > 0.0,
                    inter / jnp.where(union > 0.0, union, 1.0), 0.0)
    s_scr[...] = (iou > THR).astype(jnp.float32)

    li = lax.broadcasted_iota(jnp.int32, (1, NSEL), 1)
    keep0 = (li < PRE).astype(jnp.float32)

    def body(i, keep):
        ki = jnp.sum(keep * (li == i).astype(jnp.float32))
        row = s_scr[pl.ds(i, 1), :]
        lgt = (li > i).astype(jnp.float32)
        return keep * (1.0 - row * ki * lgt)

    keep = lax.fori_loop(0, PRE, body, keep0)

    lic = lax.broadcasted_iota(jnp.int32, (NSEL, 1), 0)
    LT = (lic <= li).astype(jnp.float32)
    incl = jnp.dot(keep, LT, preferred_element_type=jnp.float32)
    oc = lax.broadcasted_iota(jnp.int32, (104, 1), 0).astype(jnp.float32)
    ohsel = (oc == (incl - 1.0)).astype(jnp.float32) * keep
    out_ref[0] = jnp.dot(ohsel, scol, preferred_element_type=jnp.float32)


def _run_nms(ssel, isel, bc, br):
    B = ssel.shape[0]
    return pl.pallas_call(
        _nms_body,
        grid=(B,),
        in_specs=[
            pl.BlockSpec((1, 1, NSEL), lambda b: (b, 0, 0)),
            pl.BlockSpec((1, NSEL, 1), lambda b: (b, 0, 0)),
            pl.BlockSpec((1, 1, NSEL), lambda b: (b, 0, 0)),
            pl.BlockSpec((1, NSEL, 1), lambda b: (b, 0, 0)),
            pl.BlockSpec((1, NSEL, 4), lambda b: (b, 0, 0)),
            pl.BlockSpec((1, 4, NSEL), lambda b: (b, 0, 0)),
        ],
        out_specs=pl.BlockSpec((1, 104, 4), lambda b: (b, 0, 0)),
        out_shape=jax.ShapeDtypeStruct((B, 104, 4), jnp.float32),
        scratch_shapes=[pltpu.VMEM((NSEL, NSEL), jnp.float32)],
    )(ssel[:, None, :], ssel[..., None], isel[:, None, :], isel[..., None],
      bc, br)


def _gen_anchor_planes():
    base_size, ratios, scales = 16, (0.5, 1, 2), (8, 16, 32)
    anch = []
    for scale in scales:
        for ratio in ratios:
            w = base_size * scale * ratio ** 0.5
            h = base_size * scale / ratio ** 0.5
            anch.append([-w / 2, -h / 2, w / 2, h / 2])
    a = np.array(anch, dtype=np.float32)
    aw = (a[:, 2] - a[:, 0])[None, :]
    ah = (a[:, 3] - a[:, 1])[None, :]
    bcx = (a[:, 0] + 0.5 * aw[0])[None, :]
    bcy = (a[:, 1] + 0.5 * ah[0])[None, :]
    p = np.arange(1024)
    sx = ((p % 32) * 16).astype(np.float32)[:, None]
    sy = ((p // 32) * 16).astype(np.float32)[:, None]
    acx = sx + bcx
    acy = sy + bcy
    return (jnp.asarray(aw), jnp.asarray(ah),
            jnp.asarray(acx), jnp.asarray(acy))


def kernel(features, image_size, conv_w, conv_b, cls_w, cls_b, bbox_w, bbox_b):
    B, C, H, W = features.shape
    X = features.transpose(0, 2, 3, 1).reshape(B, H * W, C)
    Wt = conv_w.transpose(2, 3, 1, 0).reshape(9, C, 256)
    cw = cls_w.reshape(9, 256).T
    bw = bbox_w.reshape(9, 4, 256)
    z7 = jnp.zeros((256, 7), jnp.float32)
    W2 = jnp.concatenate([cw, z7, bw[:, 0].T, z7, bw[:, 1].T, z7,
                          bw[:, 2].T, z7, bw[:, 3].T, z7], axis=1)
    bb = bbox_b.reshape(9, 4)
    z7b = jnp.zeros((7,), jnp.float32)
    b80 = jnp.concatenate([cls_b, z7b, bb[:, 0], z7b, bb[:, 1], z7b,
                           bb[:, 2], z7b, bb[:, 3], z7b])[None, :]
    aw, ah, acx, acy = _gen_anchor_planes()
    img = image_size.astype(jnp.int32)

    sc, x1, y1, x2, y2 = _run_trunk(X, Wt, W2, conv_b[None, :], b80, img,
                                    aw, ah, acx, acy)
    scores = sc.reshape(B, H * W * 9)
    planes = [v.reshape(B, H * W * 9) for v in (x1, y1, x2, y2)]

    tau = _run_tau(scores)

    top_s, top_i = jax.vmap(lambda s: lax.top_k(s, PRE))(scores)
    selc = [jnp.take_along_axis(pl_, top_i, axis=1) for pl_ in planes]
    npad = NSEL - PRE
    ssel = jnp.concatenate(
        [top_s, jnp.full((B, npad), -np.inf, jnp.float32)], axis=1)
    isel = jnp.concatenate(
        [top_i.astype(jnp.int32),
         (1 << 22) + jnp.arange(npad, dtype=jnp.int32)[None, :]
         + jnp.zeros((B, 1), jnp.int32)], axis=1)
    bc = jnp.stack([jnp.concatenate(
        [c, jnp.zeros((B, npad), jnp.float32)], axis=1)
        for c in selc], axis=2)
    br = bc.transpose(0, 2, 1)

    out = _run_nms(ssel, isel, bc, br)[:, :POST, :]
    return out + (tau.astype(jnp.float32).sum() * 0.0)


# speedup vs baseline: 9.0730x; 1.1263x over previous
import functools
import numpy as np
import jax
import jax.numpy as jnp
from jax import lax
from jax.experimental import pallas as pl
from jax.experimental.pallas import tpu as pltpu
from jax.experimental.pallas import tpu_sc as plsc

PRE = 600
POST = 100
THR = 0.7
NSEL = 640
I32MIN = np.int32(-2147483648)
I32TOP = np.int32(0x7FFFFFFF)


def _trunk_body(x_ref, wt_ref, w2_ref, b256_ref, b80_ref, img_ref,
                aw_ref, ah_ref, acx_ref, acy_ref,
                sc_ref, x1_ref, y1_ref, x2_ref, y2_ref):
    X = x_ref[0]
    p = lax.broadcasted_iota(jnp.int32, (1024, 1), 0)
    xcol = p & 31
    acc = jnp.zeros((1024, 256), jnp.float32)
    for dy in range(3):
        for dx in range(3):
            s = (dy - 1) * 32 + (dx - 1)
            if s > 0:
                Xs = jnp.concatenate(
                    [X[s:], jnp.zeros((s, 384), jnp.float32)], axis=0)
            elif s < 0:
                Xs = jnp.concatenate(
                    [jnp.zeros((-s, 384), jnp.float32), X[:s]], axis=0)
            else:
                Xs = X
            if dx == 0:
                Xs = Xs * (xcol >= 1).astype(jnp.float32)
            elif dx == 2:
                Xs = Xs * (xcol <= 30).astype(jnp.float32)
            acc = acc + jnp.dot(Xs, wt_ref[dy * 3 + dx],
                                preferred_element_type=jnp.float32)
    t = jnp.maximum(acc + b256_ref[...], 0.0)
    out80 = jnp.dot(t, w2_ref[...],
                    preferred_element_type=jnp.float32) + b80_ref[...]
    sc = out80[:, 0:9]
    dxv = out80[:, 16:25]
    dyv = out80[:, 32:41]
    dwv = out80[:, 48:57]
    dhv = out80[:, 64:73]
    aw = aw_ref[...]
    ah = ah_ref[...]
    pcx = acx_ref[...] + dxv * aw
    pcy = acy_ref[...] + dyv * ah
    pw = aw * jnp.exp(dwv)
    ph = ah * jnp.exp(dhv)
    imh = img_ref[0].astype(jnp.float32)
    imw = img_ref[1].astype(jnp.float32)
    sc_ref[0] = sc
    x1_ref[0] = jnp.minimum(jnp.maximum(pcx - 0.5 * pw, 0.0), imw)
    y1_ref[0] = jnp.minimum(jnp.maximum(pcy - 0.5 * ph, 0.0), imh)
    x2_ref[0] = jnp.minimum(jnp.maximum(pcx + 0.5 * pw, 0.0), imw)
    y2_ref[0] = jnp.minimum(jnp.maximum(pcy + 0.5 * ph, 0.0), imh)


def _run_trunk(X, Wt, W2, b256, b80, img, aw, ah, acx, acy):
    B = X.shape[0]
    o = jax.ShapeDtypeStruct((B, 1024, 9), jnp.float32)
    full = lambda shp: pl.BlockSpec(shp, lambda b: tuple(0 for _ in shp))
    return pl.pallas_call(
        _trunk_body,
        grid=(B,),
        in_specs=[
            pl.BlockSpec((1, 1024, 384), lambda b: (b, 0, 0)),
            full((9, 384, 256)),
            full((256, 80)),
            full((1, 256)),
            full((1, 80)),
            pl.BlockSpec(memory_space=pltpu.SMEM),
            full((1, 9)),
            full((1, 9)),
            full((1024, 9)),
            full((1024, 9)),
        ],
        out_specs=[pl.BlockSpec((1, 1024, 9), lambda b: (b, 0, 0))] * 5,
        out_shape=[o] * 5,
    )(X, Wt, W2, b256, b80, img, aw, ah, acx, acy)


def _tau_body(sc_ref, tau_ref, key_ref):
    s = sc_ref[...]
    i = lax.bitcast_convert_type(s, jnp.int32)
    K = i ^ (lax.shift_right_arithmetic(i, 31) & I32TOP)
    key_ref[...] = K
    U = K ^ I32MIN

    def body(it, v):
        j = 31 - it
        cand = v | lax.shift_left(jnp.int32(1), j)
        m = ~(lax.shift_left(jnp.int32(1), j) - 1)
        a = (U & m) ^ I32MIN
        b = cand ^ I32MIN
        cnt = jnp.sum((a >= b).astype(jnp.int32), axis=1, keepdims=True)
        return jnp.where(cnt >= PRE, cand, v)

    v = lax.fori_loop(0, 32, body,
                      jnp.zeros((s.shape[0], 1), jnp.int32))
    tau_ref[...] = v ^ I32MIN


def _run_tau(scores):
    B, N = scores.shape
    return pl.pallas_call(
        _tau_body,
        out_shape=[jax.ShapeDtypeStruct((B, 1), jnp.int32),
                   jax.ShapeDtypeStruct((B, N), jnp.int32)],
    )(scores)


def _select_body(s_hbm, k_hbm, x1_hbm, y1_hbm, x2_hbm, y2_hbm, tau_hbm,
                 os_hbm, oi_hbm, ox1_hbm, oy1_hbm, ox2_hbm, oy2_hbm,
                 s_v, k_v, x1_v, y1_v, x2_v, y2_v, tau_v,
                 g_s, g_i, g_x1, g_y1, g_x2, g_y2,
                 e_s, e_i, e_x1, e_y1, e_x2, e_y2,
                 f_s, f_i, f_x1, f_y1, f_x2, f_y2):
    B = s_hbm.shape[0]
    N = s_hbm.shape[1]
    wid = lax.axis_index("s") * 2 + lax.axis_index("c")

    @pl.when(wid < B)
    def _():
        b = wid
        pltpu.sync_copy(s_hbm.at[b], s_v)
        pltpu.sync_copy(k_hbm.at[b], k_v)
        pltpu.sync_copy(x1_hbm.at[b], x1_v)
        pltpu.sync_copy(y1_hbm.at[b], y1_v)
        pltpu.sync_copy(x2_hbm.at[b], x2_v)
        pltpu.sync_copy(y2_hbm.at[b], y2_v)
        pltpu.sync_copy(tau_hbm.at[b], tau_v)
        tau = tau_v[...]
        lanes = lax.iota(jnp.int32, 16)

        def scan(v, carry):
            run_gt, run_eq = carry
            s = s_v[pl.ds(v * 16, 16)]
            key = k_v[pl.ds(v * 16, 16)]
            gt = key > tau
            eq = key == tau
            gti = gt.astype(jnp.int32)
            eqi = eq.astype(jnp.int32)
            cg = plsc.cumsum(gti)
            ce = plsc.cumsum(eqi)
            slot_g = run_gt + (cg - gti)
            slot_e = run_eq + (ce - eqi)
            idxv = v * 16 + lanes
            okg = gt & (slot_g < NSEL)
            oke = eq & (slot_e < NSEL)
            plsc.store_scatter(g_s, [slot_g], s, mask=okg)
            plsc.store_scatter(g_i, [slot_g], idxv, mask=okg)
            plsc.store_scatter(g_x1, [slot_g], x1_v[pl.ds(v * 16, 16)], mask=okg)
            plsc.store_scatter(g_y1, [slot_g], y1_v[pl.ds(v * 16, 16)], mask=okg)
            plsc.store_scatter(g_x2, [slot_g], x2_v[pl.ds(v * 16, 16)], mask=okg)
            plsc.store_scatter(g_y2, [slot_g], y2_v[pl.ds(v * 16, 16)], mask=okg)
            plsc.store_scatter(e_s, [slot_e], s, mask=oke)
            plsc.store_scatter(e_i, [slot_e], idxv, mask=oke)
            plsc.store_scatter(e_x1, [slot_e], x1_v[pl.ds(v * 16, 16)], mask=oke)
            plsc.store_scatter(e_y1, [slot_e], y1_v[pl.ds(v * 16, 16)], mask=oke)
            plsc.store_scatter(e_x2, [slot_e], x2_v[pl.ds(v * 16, 16)], mask=oke)
            plsc.store_scatter(e_y2, [slot_e], y2_v[pl.ds(v * 16, 16)], mask=oke)
            return run_gt + jnp.sum(gti), run_eq + jnp.sum(eqi)

        n_gt, n_eq = lax.fori_loop(0, N // 16, scan,
                                   (jnp.int32(0), jnp.int32(0)))

        def init(v, _):
            sl = pl.ds(v * 16, 16)
            j = v * 16 + lanes
            f_s[sl] = jnp.full((16,), -jnp.inf, jnp.float32)
            f_i[sl] = (1 << 22) + j
            z = jnp.zeros((16,), jnp.float32)
            f_x1[sl] = z
            f_y1[sl] = z
            f_x2[sl] = z
            f_y2[sl] = z
            return 0

        lax.fori_loop(0, NSEL // 16, init, 0)

        def cpg(v, _):
            sl = pl.ds(v * 16, 16)
            j = v * 16 + lanes
            m = j < n_gt
            plsc.store_scatter(f_s, [j], g_s[sl], mask=m)
            plsc.store_scatter(f_i, [j], g_i[sl], mask=m)
            plsc.store_scatter(f_x1, [j], g_x1[sl], mask=m)
            plsc.store_scatter(f_y1, [j], g_y1[sl], mask=m)
            plsc.store_scatter(f_x2, [j], g_x2[sl], mask=m)
            plsc.store_scatter(f_y2, [j], g_y2[sl], mask=m)
            return 0

        lax.fori_loop(0, NSEL // 16, cpg, 0)
        n_take = jnp.minimum(n_eq, PRE - n_gt)

        def cpe(v, _):
            sl = pl.ds(v * 16, 16)
            j = v * 16 + lanes
            m = j < n_take
            plsc.store_scatter(f_s, [n_gt + j], e_s[sl], mask=m)
            plsc.store_scatter(f_i, [n_gt + j], e_i[sl], mask=m)
            plsc.store_scatter(f_x1, [n_gt + j], e_x1[sl], mask=m)
            plsc.store_scatter(f_y1, [n_gt + j], e_y1[sl], mask=m)
            plsc.store_scatter(f_x2, [n_gt + j], e_x2[sl], mask=m)
            plsc.store_scatter(f_y2, [n_gt + j], e_y2[sl], mask=m)
            return 0

        lax.fori_loop(0, NSEL // 16, cpe, 0)
        pltpu.sync_copy(f_s, os_hbm.at[b])
        pltpu.sync_copy(f_i, oi_hbm.at[b])
        pltpu.sync_copy(f_x1, ox1_hbm.at[b])
        pltpu.sync_copy(f_y1, oy1_hbm.at[b])
        pltpu.sync_copy(f_x2, ox2_hbm.at[b])
        pltpu.sync_copy(f_y2, oy2_hbm.at[b])


def _run_select(scores, keys, x1, y1, x2, y2, tau16):
    B, N = scores.shape
    mesh = plsc.VectorSubcoreMesh(core_axis_name="c", subcore_axis_name="s")
    fo = jax.ShapeDtypeStruct((B, NSEL), jnp.float32)
    io = jax.ShapeDtypeStruct((B, NSEL), jnp.int32)
    vN = lambda dt: pltpu.VMEM((N,), dt)
    vS = lambda dt: pltpu.VMEM((NSEL,), dt)
    fn = pl.kernel(
        _select_body, mesh=mesh,
        compiler_params=pltpu.CompilerParams(needs_layout_passes=False),
        out_type=[fo, io, fo, fo, fo, fo],
        scratch_types=[
            vN(jnp.float32), vN(jnp.int32), vN(jnp.float32), vN(jnp.float32),
            vN(jnp.float32), vN(jnp.float32), pltpu.VMEM((16,), jnp.int32),
            vS(jnp.float32), vS(jnp.int32), vS(jnp.float32), vS(jnp.float32),
            vS(jnp.float32), vS(jnp.float32),
            vS(jnp.float32), vS(jnp.int32), vS(jnp.float32), vS(jnp.float32),
            vS(jnp.float32), vS(jnp.float32),
            vS(jnp.float32), vS(jnp.int32), vS(jnp.float32), vS(jnp.float32),
            vS(jnp.float32), vS(jnp.float32),
        ],
    )
    return fn(scores, keys, x1, y1, x2, y2, tau16)


def _nms_body(sr_ref, scv_ref, ir_ref, icv_ref, bc_ref, br_ref,
              out_ref, s_scr):
    sr = sr_ref[0]
    scv = scv_ref[0]
    ir = ir_ref[0]
    icv = icv_ref[0]
    bc = bc_ref[0]
    br = br_ref[0]

    beats_ji = (sr > scv) | ((sr == scv) & (ir < icv))
    rank_col = jnp.sum(beats_ji.astype(jnp.float32), axis=1, keepdims=True)
    beats_ij = (scv > sr) | ((scv == sr) & (icv < ir))
    rank_row = jnp.sum(beats_ij.astype(jnp.float32), axis=0, keepdims=True)

    r_c = lax.broadcasted_iota(jnp.int32, (NSEL, 1), 0).astype(jnp.float32)
    r_r = lax.broadcasted_iota(jnp.int32, (1, NSEL), 1).astype(jnp.float32)
    P = (r_c == rank_row).astype(jnp.float32)
    PT = (rank_col == r_r).astype(jnp.float32)
    scol = jnp.dot(P, bc, preferred_element_type=jnp.float32)
    srow = jnp.dot(br, PT, preferred_element_type=jnp.float32)

    x1c = scol[:, 0:1]
    y1c = scol[:, 1:2]
    x2c = scol[:, 2:3]
    y2c = scol[:, 3:4]
    x1r = srow[0:1, :]
    y1r = srow[1:2, :]
    x2r = srow[2:3, :]
    y2r = srow[3:4, :]
    areas_c = (x2c - x1c) * (y2c - y1c)
    areas_r = (x2r - x1r) * (y2r - y1r)
    xx1 = jnp.maximum(x1c, x1r)
    yy1 = jnp.maximum(y1c, y1r)
    xx2 = jnp.minimum(x2c, x2r)
    yy2 = jnp.minimum(y2c, y2r)
    inter = jnp.maximum(xx2 - xx1, 0.0) * jnp.maximum(yy2 - yy1, 0.0)
    union = areas_c + areas_r - inter
    iou = jnp.where(union > 0.0,
                    inter / jnp.where(union > 0.0, union, 1.0), 0.0)
    s_scr[...] = (iou > THR).astype(jnp.float32)

    li = lax.broadcasted_iota(jnp.int32, (1, NSEL), 1)
    keep0 = (li < PRE).astype(jnp.float32)

    def body(i, keep):
        ki = jnp.sum(keep * (li == i).astype(jnp.float32))
        row = s_scr[pl.ds(i, 1), :]
        lgt = (li > i).astype(jnp.float32)
        return keep * (1.0 - row * ki * lgt)

    keep = lax.fori_loop(0, PRE, body, keep0)

    lic = lax.broadcasted_iota(jnp.int32, (NSEL, 1), 0)
    LT = (lic <= li).astype(jnp.float32)
    incl = jnp.dot(keep, LT, preferred_element_type=jnp.float32)
    oc = lax.broadcasted_iota(jnp.int32, (104, 1), 0).astype(jnp.float32)
    ohsel = (oc == (incl - 1.0)).astype(jnp.float32) * keep
    out_ref[0] = jnp.dot(ohsel, scol, preferred_element_type=jnp.float32)


def _run_nms(ssel, isel, bc, br):
    B = ssel.shape[0]
    return pl.pallas_call(
        _nms_body,
        grid=(B,),
        in_specs=[
            pl.BlockSpec((1, 1, NSEL), lambda b: (b, 0, 0)),
            pl.BlockSpec((1, NSEL, 1), lambda b: (b, 0, 0)),
            pl.BlockSpec((1, 1, NSEL), lambda b: (b, 0, 0)),
            pl.BlockSpec((1, NSEL, 1), lambda b: (b, 0, 0)),
            pl.BlockSpec((1, NSEL, 4), lambda b: (b, 0, 0)),
            pl.BlockSpec((1, 4, NSEL), lambda b: (b, 0, 0)),
        ],
        out_specs=pl.BlockSpec((1, 104, 4), lambda b: (b, 0, 0)),
        out_shape=jax.ShapeDtypeStruct((B, 104, 4), jnp.float32),
        scratch_shapes=[pltpu.VMEM((NSEL, NSEL), jnp.float32)],
    )(ssel[:, None, :], ssel[..., None], isel[:, None, :], isel[..., None],
      bc, br)


def _gen_anchor_planes():
    base_size, ratios, scales = 16, (0.5, 1, 2), (8, 16, 32)
    anch = []
    for scale in scales:
        for ratio in ratios:
            w = base_size * scale * ratio ** 0.5
            h = base_size * scale / ratio ** 0.5
            anch.append([-w / 2, -h / 2, w / 2, h / 2])
    a = np.array(anch, dtype=np.float32)
    aw = (a[:, 2] - a[:, 0])[None, :]
    ah = (a[:, 3] - a[:, 1])[None, :]
    bcx = (a[:, 0] + 0.5 * aw[0])[None, :]
    bcy = (a[:, 1] + 0.5 * ah[0])[None, :]
    p = np.arange(1024)
    sx = ((p % 32) * 16).astype(np.float32)[:, None]
    sy = ((p // 32) * 16).astype(np.float32)[:, None]
    acx = sx + bcx
    acy = sy + bcy
    return (jnp.asarray(aw), jnp.asarray(ah),
            jnp.asarray(acx), jnp.asarray(acy))


def kernel(features, image_size, conv_w, conv_b, cls_w, cls_b, bbox_w, bbox_b):
    B, C, H, W = features.shape
    X = features.transpose(0, 2, 3, 1).reshape(B, H * W, C)
    Wt = conv_w.transpose(2, 3, 1, 0).reshape(9, C, 256)
    cw = cls_w.reshape(9, 256).T
    bw = bbox_w.reshape(9, 4, 256)
    z7 = jnp.zeros((256, 7), jnp.float32)
    W2 = jnp.concatenate([cw, z7, bw[:, 0].T, z7, bw[:, 1].T, z7,
                          bw[:, 2].T, z7, bw[:, 3].T, z7], axis=1)
    bb = bbox_b.reshape(9, 4)
    z7b = jnp.zeros((7,), jnp.float32)
    b80 = jnp.concatenate([cls_b, z7b, bb[:, 0], z7b, bb[:, 1], z7b,
                           bb[:, 2], z7b, bb[:, 3], z7b])[None, :]
    aw, ah, acx, acy = _gen_anchor_planes()
    img = image_size.astype(jnp.int32)

    sc, x1, y1, x2, y2 = _run_trunk(X, Wt, W2, conv_b[None, :], b80, img,
                                    aw, ah, acx, acy)
    scores = sc.reshape(B, H * W * 9)
    planes = [v.reshape(B, H * W * 9) for v in (x1, y1, x2, y2)]

    tau, keys = _run_tau(scores)
    tau16 = jnp.broadcast_to(tau, (B, 16))

    ssel, isel, sx1, sy1, sx2, sy2 = _run_select(
        scores, keys, planes[0], planes[1], planes[2], planes[3], tau16)
    bc = jnp.stack([sx1, sy1, sx2, sy2], axis=2)
    br = bc.transpose(0, 2, 1)

    return _run_nms(ssel, isel, bc, br)[:, :POST, :]


# speedup vs baseline: 21.6512x; 2.3863x over previous
import functools
import numpy as np
import jax
import jax.numpy as jnp
from jax import lax
from jax.experimental import pallas as pl
from jax.experimental.pallas import tpu as pltpu
from jax.experimental.pallas import tpu_sc as plsc

PRE = 600
POST = 100
THR = 0.7
NSEL = 640
I32MIN = np.int32(-2147483648)
I32TOP = np.int32(0x7FFFFFFF)


def _trunk_body(x_ref, wt_ref, w2_ref, b256_ref, b80_ref, img_ref,
                aw_ref, ah_ref, acx_ref, acy_ref,
                sc_ref, x1_ref, y1_ref, x2_ref, y2_ref):
    X = x_ref[0]
    p = lax.broadcasted_iota(jnp.int32, (1024, 1), 0)
    xcol = p & 31
    acc = jnp.zeros((1024, 256), jnp.float32)
    for dy in range(3):
        for dx in range(3):
            s = (dy - 1) * 32 + (dx - 1)
            if s > 0:
                Xs = jnp.concatenate(
                    [X[s:], jnp.zeros((s, 384), jnp.float32)], axis=0)
            elif s < 0:
                Xs = jnp.concatenate(
                    [jnp.zeros((-s, 384), jnp.float32), X[:s]], axis=0)
            else:
                Xs = X
            if dx == 0:
                Xs = Xs * (xcol >= 1).astype(jnp.float32)
            elif dx == 2:
                Xs = Xs * (xcol <= 30).astype(jnp.float32)
            acc = acc + jnp.dot(Xs, wt_ref[dy * 3 + dx],
                                preferred_element_type=jnp.float32)
    t = jnp.maximum(acc + b256_ref[...], 0.0)
    out80 = jnp.dot(t, w2_ref[...],
                    preferred_element_type=jnp.float32) + b80_ref[...]
    sc = out80[:, 0:9]
    dxv = out80[:, 16:25]
    dyv = out80[:, 32:41]
    dwv = out80[:, 48:57]
    dhv = out80[:, 64:73]
    aw = aw_ref[...]
    ah = ah_ref[...]
    pcx = acx_ref[...] + dxv * aw
    pcy = acy_ref[...] + dyv * ah
    pw = aw * jnp.exp(dwv)
    ph = ah * jnp.exp(dhv)
    imh = img_ref[0].astype(jnp.float32)
    imw = img_ref[1].astype(jnp.float32)
    sc_ref[0] = sc
    x1_ref[0] = jnp.minimum(jnp.maximum(pcx - 0.5 * pw, 0.0), imw)
    y1_ref[0] = jnp.minimum(jnp.maximum(pcy - 0.5 * ph, 0.0), imh)
    x2_ref[0] = jnp.minimum(jnp.maximum(pcx + 0.5 * pw, 0.0), imw)
    y2_ref[0] = jnp.minimum(jnp.maximum(pcy + 0.5 * ph, 0.0), imh)


def _run_trunk(X, Wt, W2, b256, b80, img, aw, ah, acx, acy):
    B = X.shape[0]
    o = jax.ShapeDtypeStruct((B, 1024, 9), jnp.float32)
    full = lambda shp: pl.BlockSpec(shp, lambda b: tuple(0 for _ in shp))
    return pl.pallas_call(
        _trunk_body,
        grid=(B,),
        in_specs=[
            pl.BlockSpec((1, 1024, 384), lambda b: (b, 0, 0)),
            full((9, 384, 256)),
            full((256, 80)),
            full((1, 256)),
            full((1, 80)),
            pl.BlockSpec(memory_space=pltpu.SMEM),
            full((1, 9)),
            full((1, 9)),
            full((1024, 9)),
            full((1024, 9)),
        ],
        out_specs=[pl.BlockSpec((1, 1024, 9), lambda b: (b, 0, 0))] * 5,
        out_shape=[o] * 5,
    )(X, Wt, W2, b256, b80, img, aw, ah, acx, acy)


def _tau_body(sc_ref, tau_ref, key_ref):
    s = sc_ref[...]
    i = lax.bitcast_convert_type(s, jnp.int32)
    K = i ^ (lax.shift_right_arithmetic(i, 31) & I32TOP)
    key_ref[...] = K
    U = K ^ I32MIN

    def body(it, v):
        j = 31 - it
        cand = v | lax.shift_left(jnp.int32(1), j)
        m = ~(lax.shift_left(jnp.int32(1), j) - 1)
        a = (U & m) ^ I32MIN
        b = cand ^ I32MIN
        cnt = jnp.sum((a >= b).astype(jnp.int32), axis=1, keepdims=True)
        return jnp.where(cnt >= PRE, cand, v)

    v = lax.fori_loop(0, 32, body,
                      jnp.zeros((s.shape[0], 1), jnp.int32))
    tau_ref[...] = v ^ I32MIN


def _run_tau(scores):
    B, N = scores.shape
    return pl.pallas_call(
        _tau_body,
        out_shape=[jax.ShapeDtypeStruct((B, 1), jnp.int32),
                   jax.ShapeDtypeStruct((B, N), jnp.int32)],
    )(scores)


def _select_body(s_hbm, k_hbm, x1_hbm, y1_hbm, x2_hbm, y2_hbm, tau_hbm,
                 os_hbm, oi_hbm, ox1_hbm, oy1_hbm, ox2_hbm, oy2_hbm,
                 s_v, k_v, x1_v, y1_v, x2_v, y2_v, tau_v,
                 g_s, g_i, g_x1, g_y1, g_x2, g_y2,
                 e_s, e_i, e_x1, e_y1, e_x2, e_y2,
                 f_s, f_i, f_x1, f_y1, f_x2, f_y2):
    B = s_hbm.shape[0]
    N = s_hbm.shape[1]
    wid = lax.axis_index("s") * 2 + lax.axis_index("c")

    @pl.when(wid < B)
    def _():
        b = wid
        pltpu.sync_copy(s_hbm.at[b], s_v)
        pltpu.sync_copy(k_hbm.at[b], k_v)
        pltpu.sync_copy(x1_hbm.at[b], x1_v)
        pltpu.sync_copy(y1_hbm.at[b], y1_v)
        pltpu.sync_copy(x2_hbm.at[b], x2_v)
        pltpu.sync_copy(y2_hbm.at[b], y2_v)
        pltpu.sync_copy(tau_hbm.at[b], tau_v)
        tau = tau_v[...]
        lanes = lax.iota(jnp.int32, 16)

        def scan(v, carry):
            run_gt, run_eq = carry
            s = s_v[pl.ds(v * 16, 16)]
            key = k_v[pl.ds(v * 16, 16)]
            gt = key > tau
            eq = key == tau
            gti = gt.astype(jnp.int32)
            eqi = eq.astype(jnp.int32)
            cg = plsc.cumsum(gti)
            ce = plsc.cumsum(eqi)
            slot_g = run_gt + (cg - gti)
            slot_e = run_eq + (ce - eqi)
            idxv = v * 16 + lanes
            okg = gt & (slot_g < NSEL)
            oke = eq & (slot_e < NSEL)
            plsc.store_scatter(g_s, [slot_g], s, mask=okg)
            plsc.store_scatter(g_i, [slot_g], idxv, mask=okg)
            plsc.store_scatter(g_x1, [slot_g], x1_v[pl.ds(v * 16, 16)], mask=okg)
            plsc.store_scatter(g_y1, [slot_g], y1_v[pl.ds(v * 16, 16)], mask=okg)
            plsc.store_scatter(g_x2, [slot_g], x2_v[pl.ds(v * 16, 16)], mask=okg)
            plsc.store_scatter(g_y2, [slot_g], y2_v[pl.ds(v * 16, 16)], mask=okg)
            plsc.store_scatter(e_s, [slot_e], s, mask=oke)
            plsc.store_scatter(e_i, [slot_e], idxv, mask=oke)
            plsc.store_scatter(e_x1, [slot_e], x1_v[pl.ds(v * 16, 16)], mask=oke)
            plsc.store_scatter(e_y1, [slot_e], y1_v[pl.ds(v * 16, 16)], mask=oke)
            plsc.store_scatter(e_x2, [slot_e], x2_v[pl.ds(v * 16, 16)], mask=oke)
            plsc.store_scatter(e_y2, [slot_e], y2_v[pl.ds(v * 16, 16)], mask=oke)
            return run_gt + jnp.sum(gti), run_eq + jnp.sum(eqi)

        n_gt, n_eq = lax.fori_loop(0, N // 16, scan,
                                   (jnp.int32(0), jnp.int32(0)))

        def init(v, _):
            sl = pl.ds(v * 16, 16)
            j = v * 16 + lanes
            f_s[sl] = jnp.full((16,), -jnp.inf, jnp.float32)
            f_i[sl] = (1 << 22) + j
            z = jnp.zeros((16,), jnp.float32)
            f_x1[sl] = z
            f_y1[sl] = z
            f_x2[sl] = z
            f_y2[sl] = z
            return 0

        lax.fori_loop(0, NSEL // 16, init, 0)

        def cpg(v, _):
            sl = pl.ds(v * 16, 16)
            j = v * 16 + lanes
            m = j < n_gt
            plsc.store_scatter(f_s, [j], g_s[sl], mask=m)
            plsc.store_scatter(f_i, [j], g_i[sl], mask=m)
            plsc.store_scatter(f_x1, [j], g_x1[sl], mask=m)
            plsc.store_scatter(f_y1, [j], g_y1[sl], mask=m)
            plsc.store_scatter(f_x2, [j], g_x2[sl], mask=m)
            plsc.store_scatter(f_y2, [j], g_y2[sl], mask=m)
            return 0

        lax.fori_loop(0, NSEL // 16, cpg, 0)
        n_take = jnp.minimum(n_eq, PRE - n_gt)

        def cpe(v, _):
            sl = pl.ds(v * 16, 16)
            j = v * 16 + lanes
            m = j < n_take
            plsc.store_scatter(f_s, [n_gt + j], e_s[sl], mask=m)
            plsc.store_scatter(f_i, [n_gt + j], e_i[sl], mask=m)
            plsc.store_scatter(f_x1, [n_gt + j], e_x1[sl], mask=m)
            plsc.store_scatter(f_y1, [n_gt + j], e_y1[sl], mask=m)
            plsc.store_scatter(f_x2, [n_gt + j], e_x2[sl], mask=m)
            plsc.store_scatter(f_y2, [n_gt + j], e_y2[sl], mask=m)
            return 0

        lax.fori_loop(0, NSEL // 16, cpe, 0)
        pltpu.sync_copy(f_s, os_hbm.at[b])
        pltpu.sync_copy(f_i, oi_hbm.at[b])
        pltpu.sync_copy(f_x1, ox1_hbm.at[b])
        pltpu.sync_copy(f_y1, oy1_hbm.at[b])
        pltpu.sync_copy(f_x2, ox2_hbm.at[b])
        pltpu.sync_copy(f_y2, oy2_hbm.at[b])


def _run_select(scores, keys, x1, y1, x2, y2, tau16):
    B, N = scores.shape
    mesh = plsc.VectorSubcoreMesh(core_axis_name="c", subcore_axis_name="s")
    fo = jax.ShapeDtypeStruct((B, NSEL), jnp.float32)
    io = jax.ShapeDtypeStruct((B, NSEL), jnp.int32)
    vN = lambda dt: pltpu.VMEM((N,), dt)
    vS = lambda dt: pltpu.VMEM((NSEL,), dt)
    fn = pl.kernel(
        _select_body, mesh=mesh,
        compiler_params=pltpu.CompilerParams(needs_layout_passes=False),
        out_type=[fo, io, fo, fo, fo, fo],
        scratch_types=[
            vN(jnp.float32), vN(jnp.int32), vN(jnp.float32), vN(jnp.float32),
            vN(jnp.float32), vN(jnp.float32), pltpu.VMEM((16,), jnp.int32),
            vS(jnp.float32), vS(jnp.int32), vS(jnp.float32), vS(jnp.float32),
            vS(jnp.float32), vS(jnp.float32),
            vS(jnp.float32), vS(jnp.int32), vS(jnp.float32), vS(jnp.float32),
            vS(jnp.float32), vS(jnp.float32),
            vS(jnp.float32), vS(jnp.int32), vS(jnp.float32), vS(jnp.float32),
            vS(jnp.float32), vS(jnp.float32),
        ],
    )
    return fn(scores, keys, x1, y1, x2, y2, tau16)


def _nms_body(sr_ref, scv_ref, ir_ref, icv_ref, bc_ref, br_ref,
              out_ref, s_scr, sb_scr):
    b = pl.program_id(0)
    nb = pl.num_programs(0)
    sr = sr_ref[0]
    scv = scv_ref[0]
    ir = ir_ref[0]
    icv = icv_ref[0]
    bc = bc_ref[0]
    br = br_ref[0]

    beats_ji = (sr > scv) | ((sr == scv) & (ir < icv))
    rank_col = jnp.sum(beats_ji.astype(jnp.float32), axis=1, keepdims=True)
    beats_ij = (scv > sr) | ((scv == sr) & (icv < ir))
    rank_row = jnp.sum(beats_ij.astype(jnp.float32), axis=0, keepdims=True)

    r_c = lax.broadcasted_iota(jnp.int32, (NSEL, 1), 0).astype(jnp.float32)
    r_r = lax.broadcasted_iota(jnp.int32, (1, NSEL), 1).astype(jnp.float32)
    P = (r_c == rank_row).astype(jnp.float32)
    PT = (rank_col == r_r).astype(jnp.float32)
    scol = jnp.dot(P, bc, preferred_element_type=jnp.float32)
    srow = jnp.dot(br, PT, preferred_element_type=jnp.float32)

    x1c = scol[:, 0:1]
    y1c = scol[:, 1:2]
    x2c = scol[:, 2:3]
    y2c = scol[:, 3:4]
    x1r = srow[0:1, :]
    y1r = srow[1:2, :]
    x2r = srow[2:3, :]
    y2r = srow[3:4, :]
    areas_c = (x2c - x1c) * (y2c - y1c)
    areas_r = (x2r - x1r) * (y2r - y1r)
    xx1 = jnp.maximum(x1c, x1r)
    yy1 = jnp.maximum(y1c, y1r)
    xx2 = jnp.minimum(x2c, x2r)
    yy2 = jnp.minimum(y2c, y2r)
    inter = jnp.maximum(xx2 - xx1, 0.0) * jnp.maximum(yy2 - yy1, 0.0)
    union = areas_c + areas_r - inter
    iou = jnp.where(union > 0.0,
                    inter / jnp.where(union > 0.0, union, 1.0), 0.0)
    s_scr[b] = (iou > THR).astype(jnp.float32)
    sb_scr[b] = scol

    @pl.when(b == nb - 1)
    def _():
        li = lax.broadcasted_iota(jnp.int32, (1, NSEL), 1)
        keep0 = jnp.broadcast_to((li < PRE).astype(jnp.float32), (4, NSEL))

        def body(i, keep):
            oh = (li == i).astype(jnp.float32)
            ki = jnp.sum(keep * oh, axis=1, keepdims=True)
            row = s_scr[:, i, :]
            lgt = (li > i).astype(jnp.float32)
            return keep * (1.0 - row * ki * lgt)

        keep = lax.fori_loop(0, PRE, body, keep0)

        lic = lax.broadcasted_iota(jnp.int32, (NSEL, 1), 0)
        LT = (lic <= li).astype(jnp.float32)
        incl = jnp.dot(keep, LT,
                       preferred_element_type=jnp.float32)
        oc = lax.broadcasted_iota(jnp.int32, (104, 1), 0).astype(jnp.float32)
        for bb in range(4):
            ohsel = ((oc == (incl[bb:bb + 1] - 1.0)).astype(jnp.float32)
                     * keep[bb:bb + 1])
            out_ref[bb] = jnp.dot(ohsel, sb_scr[bb],
                                  preferred_element_type=jnp.float32)


def _run_nms(ssel, isel, bc, br):
    B = ssel.shape[0]
    return pl.pallas_call(
        _nms_body,
        grid=(B,),
        in_specs=[
            pl.BlockSpec((1, 1, NSEL), lambda b: (b, 0, 0)),
            pl.BlockSpec((1, NSEL, 1), lambda b: (b, 0, 0)),
            pl.BlockSpec((1, 1, NSEL), lambda b: (b, 0, 0)),
            pl.BlockSpec((1, NSEL, 1), lambda b: (b, 0, 0)),
            pl.BlockSpec((1, NSEL, 4), lambda b: (b, 0, 0)),
            pl.BlockSpec((1, 4, NSEL), lambda b: (b, 0, 0)),
        ],
        out_specs=pl.BlockSpec((B, 104, 4), lambda b: (0, 0, 0)),
        out_shape=jax.ShapeDtypeStruct((B, 104, 4), jnp.float32),
        scratch_shapes=[pltpu.VMEM((B, NSEL, NSEL), jnp.float32),
                        pltpu.VMEM((B, NSEL, 4), jnp.float32)],
    )(ssel[:, None, :], ssel[..., None], isel[:, None, :], isel[..., None],
      bc, br)


def _gen_anchor_planes():
    base_size, ratios, scales = 16, (0.5, 1, 2), (8, 16, 32)
    anch = []
    for scale in scales:
        for ratio in ratios:
            w = base_size * scale * ratio ** 0.5
            h = base_size * scale / ratio ** 0.5
            anch.append([-w / 2, -h / 2, w / 2, h / 2])
    a = np.array(anch, dtype=np.float32)
    aw = (a[:, 2] - a[:, 0])[None, :]
    ah = (a[:, 3] - a[:, 1])[None, :]
    bcx = (a[:, 0] + 0.5 * aw[0])[None, :]
    bcy = (a[:, 1] + 0.5 * ah[0])[None, :]
    p = np.arange(1024)
    sx = ((p % 32) * 16).astype(np.float32)[:, None]
    sy = ((p // 32) * 16).astype(np.float32)[:, None]
    acx = sx + bcx
    acy = sy + bcy
    return (jnp.asarray(aw), jnp.asarray(ah),
            jnp.asarray(acx), jnp.asarray(acy))


def kernel(features, image_size, conv_w, conv_b, cls_w, cls_b, bbox_w, bbox_b):
    B, C, H, W = features.shape
    X = features.transpose(0, 2, 3, 1).reshape(B, H * W, C)
    Wt = conv_w.transpose(2, 3, 1, 0).reshape(9, C, 256)
    cw = cls_w.reshape(9, 256).T
    bw = bbox_w.reshape(9, 4, 256)
    z7 = jnp.zeros((256, 7), jnp.float32)
    W2 = jnp.concatenate([cw, z7, bw[:, 0].T, z7, bw[:, 1].T, z7,
                          bw[:, 2].T, z7, bw[:, 3].T, z7], axis=1)
    bb = bbox_b.reshape(9, 4)
    z7b = jnp.zeros((7,), jnp.float32)
    b80 = jnp.concatenate([cls_b, z7b, bb[:, 0], z7b, bb[:, 1], z7b,
                           bb[:, 2], z7b, bb[:, 3], z7b])[None, :]
    aw, ah, acx, acy = _gen_anchor_planes()
    img = image_size.astype(jnp.int32)

    sc, x1, y1, x2, y2 = _run_trunk(X, Wt, W2, conv_b[None, :], b80, img,
                                    aw, ah, acx, acy)
    scores = sc.reshape(B, H * W * 9)
    planes = [v.reshape(B, H * W * 9) for v in (x1, y1, x2, y2)]

    tau, keys = _run_tau(scores)
    tau16 = jnp.broadcast_to(tau, (B, 16))

    ssel, isel, sx1, sy1, sx2, sy2 = _run_select(
        scores, keys, planes[0], planes[1], planes[2], planes[3], tau16)
    bc = jnp.stack([sx1, sy1, sx2, sy2], axis=2)
    br = bc.transpose(0, 2, 1)

    return _run_nms(ssel, isel, bc, br)[:, :POST, :]
